# Initial kernel scaffold; baseline (speedup 1.0000x reference)
import functools

import jax
import jax.numpy as jnp
from jax import lax
from jax.experimental import pallas as pl
from jax.experimental.pallas import tpu as pltpu
from jax.experimental.pallas import tpu_sc as plsc

N_CORES = 2
N_SUBCORES = 16
N_TILES = N_CORES * N_SUBCORES
CHUNK = 80
F = 128
CNT_W = 16


def _sc_aggregate(h, src, dst):
    n_nodes = h.shape[0]
    n_edges = src.shape[0]
    per_tile = n_edges // N_TILES
    n_chunks = per_tile // CHUNK
    rows_per_tile = n_nodes // N_SUBCORES
    zrows = rows_per_tile // 5

    mesh = plsc.VectorSubcoreMesh(core_axis_name="c", subcore_axis_name="s")

    @functools.partial(
        pl.kernel,
        out_type=[
            jax.ShapeDtypeStruct((N_CORES, n_nodes, F), jnp.float32),
            jax.ShapeDtypeStruct((N_CORES, n_nodes, CNT_W), jnp.float32),
        ],
        mesh=mesh,
        scratch_types=[
            pltpu.VMEM((CHUNK,), jnp.int32),
            pltpu.VMEM((CHUNK,), jnp.int32),
            pltpu.VMEM((CHUNK, F), jnp.float32),
            pltpu.VMEM((CHUNK, CNT_W), jnp.float32),
            pltpu.VMEM((zrows, F), jnp.float32),
            pltpu.VMEM((zrows, CNT_W), jnp.float32),
            pltpu.VMEM_SHARED((n_nodes, F), jnp.float32),
            pltpu.VMEM_SHARED((n_nodes, CNT_W), jnp.float32),
            pltpu.SemaphoreType.DMA,
        ],
    )
    def agg(h_hbm, src_hbm, dst_hbm, acc_hbm, cnt_hbm,
            src_v, dst_v, rows_v, ones_v, zrow_v, zcnt_v, acc_sh, cnt_sh, sem):
        c = lax.axis_index("c")
        s = lax.axis_index("s")
        base = (c * N_SUBCORES + s) * per_tile

        @pl.loop(0, CHUNK)
        def _(i):
            ones_v[i, :] = jnp.full((CNT_W,), 1.0, jnp.float32)

        @pl.loop(0, zrows)
        def _(i):
            for j in range(F // 16):
                zrow_v[i, pl.ds(j * 16, 16)] = jnp.zeros((16,), jnp.float32)
            zcnt_v[i, :] = jnp.zeros((CNT_W,), jnp.float32)

        for j in range(rows_per_tile // zrows):
            r0 = s * rows_per_tile + j * zrows
            pltpu.sync_copy(zrow_v, acc_sh.at[pl.ds(r0, zrows)])
            pltpu.sync_copy(zcnt_v, cnt_sh.at[pl.ds(r0, zrows)])
        plsc.subcore_barrier()

        @pl.loop(0, n_chunks)
        def _(i):
            e0 = base + i * CHUNK
            pltpu.sync_copy(src_hbm.at[pl.ds(e0, CHUNK)], src_v)
            pltpu.sync_copy(dst_hbm.at[pl.ds(e0, CHUNK)], dst_v)
            pltpu.async_copy(h_hbm.at[src_v], rows_v, sem).wait()
            pltpu.sync_copy(rows_v, acc_sh.at[dst_v], add=True)
            pltpu.sync_copy(ones_v, cnt_sh.at[dst_v], add=True)

        plsc.subcore_barrier()

        r0 = s * rows_per_tile
        pltpu.sync_copy(acc_sh.at[pl.ds(r0, rows_per_tile)],
                        acc_hbm.at[c, pl.ds(r0, rows_per_tile)])
        pltpu.sync_copy(cnt_sh.at[pl.ds(r0, rows_per_tile)],
                        cnt_hbm.at[c, pl.ds(r0, rows_per_tile)])

    return agg(h, src, dst)


def _tc_combine(h, acc, cnt, w1t, w2t, b2):
    n = h.shape[0]
    br = 1000
    grid = (n // br,)

    def body(h_ref, acc_ref, cnt_ref, w1_ref, w2_ref, b_ref, o_ref):
        a = acc_ref[0] + acc_ref[1]
        c = cnt_ref[0] + cnt_ref[1]
        inv = 1.0 / jnp.maximum(c[:, 0:1], 1.0)
        hn = a * inv
        t1 = jnp.dot(h_ref[...], w1_ref[...], preferred_element_type=jnp.float32)
        t2 = jnp.dot(hn, w2_ref[...], preferred_element_type=jnp.float32)
        o_ref[...] = t1 + t2 + b_ref[...]

    return pl.pallas_call(
        body,
        grid=grid,
        in_specs=[
            pl.BlockSpec((br, F), lambda i: (i, 0)),
            pl.BlockSpec((N_CORES, br, F), lambda i: (0, i, 0)),
            pl.BlockSpec((N_CORES, br, CNT_W), lambda i: (0, i, 0)),
            pl.BlockSpec((F, F), lambda i: (0, 0)),
            pl.BlockSpec((F, F), lambda i: (0, 0)),
            pl.BlockSpec((1, F), lambda i: (0, 0)),
        ],
        out_specs=pl.BlockSpec((br, F), lambda i: (i, 0)),
        out_shape=jax.ShapeDtypeStruct((n, F), jnp.float32),
    )(h, acc, cnt, w1t, w2t, b2)


def kernel(h, edge_index, W, b):
    src = edge_index[0]
    dst = edge_index[1]
    f_in = h.shape[1]
    w1t = W[:, :f_in].T
    w2t = W[:, f_in:].T
    b2 = b.reshape(1, -1)
    acc, cnt = _sc_aggregate(h, src, dst)
    return _tc_combine(h, acc, cnt, w1t, w2t, b2)


# speedup vs baseline: 3.9458x; 3.9458x over previous
import functools

import jax
import jax.numpy as jnp
from jax import lax
from jax.experimental import pallas as pl
from jax.experimental.pallas import tpu as pltpu
from jax.experimental.pallas import tpu_sc as plsc

N_CORES = 2
N_SUBCORES = 16
CHUNK = 80
F = 128
FH = F // 2
CNT_W = 16


def _sc_aggregate(h_lo, h_hi, src, dst):
    n_nodes = h_lo.shape[0]
    n_edges = src.shape[0]
    per_tile = n_edges // N_SUBCORES
    n_chunks = per_tile // CHUNK
    n_pad = ((n_nodes + 40 * N_SUBCORES - 1) // (40 * N_SUBCORES)) * 40 * N_SUBCORES
    rows_per_tile = n_pad // N_SUBCORES
    zrows = rows_per_tile // 5

    mesh = plsc.VectorSubcoreMesh(core_axis_name="c", subcore_axis_name="s")

    @functools.partial(
        pl.kernel,
        out_type=[
            jax.ShapeDtypeStruct((N_CORES, n_pad, FH), jnp.float32),
            jax.ShapeDtypeStruct((n_pad, CNT_W), jnp.float32),
        ],
        mesh=mesh,
        scratch_types=[
            pltpu.VMEM((CHUNK,), jnp.int32),
            pltpu.VMEM((CHUNK,), jnp.int32),
            pltpu.VMEM((CHUNK, FH), jnp.float32),
            pltpu.VMEM((CHUNK, CNT_W), jnp.float32),
            pltpu.VMEM((zrows, FH), jnp.float32),
            pltpu.VMEM((zrows, CNT_W), jnp.float32),
            pltpu.VMEM_SHARED((n_pad, FH), jnp.float32),
            pltpu.VMEM_SHARED((n_pad, CNT_W), jnp.float32),
            pltpu.SemaphoreType.DMA,
        ],
        compiler_params=pltpu.CompilerParams(use_tc_tiling_on_sc=False),
    )
    def agg(hlo_hbm, hhi_hbm, src_hbm, dst_hbm, acc_hbm, cnt_hbm,
            src_v, dst_v, rows_v, ones_v, zrow_v, zcnt_v, acc_sh, cnt_sh, sem):
        c = lax.axis_index("c")
        s = lax.axis_index("s")
        base = s * per_tile

        @pl.loop(0, CHUNK)
        def _(i):
            ones_v[i, :] = jnp.full((CNT_W,), 1.0, jnp.float32)

        @pl.loop(0, zrows)
        def _(i):
            for j in range(FH // 16):
                zrow_v[i, pl.ds(j * 16, 16)] = jnp.zeros((16,), jnp.float32)
            zcnt_v[i, :] = jnp.zeros((CNT_W,), jnp.float32)

        for j in range(rows_per_tile // zrows):
            r0 = s * rows_per_tile + j * zrows
            pltpu.sync_copy(zrow_v, acc_sh.at[pl.ds(r0, zrows)])
            pltpu.sync_copy(zcnt_v, cnt_sh.at[pl.ds(r0, zrows)])
        plsc.subcore_barrier()

        def edge_loop(h_half_hbm, with_counts):
            @pl.loop(0, n_chunks)
            def _(i):
                e0 = base + i * CHUNK
                pltpu.sync_copy(src_hbm.at[pl.ds(e0, CHUNK)], src_v)
                pltpu.sync_copy(dst_hbm.at[pl.ds(e0, CHUNK)], dst_v)
                pltpu.async_copy(h_half_hbm.at[src_v], rows_v, sem).wait()
                pltpu.sync_copy(rows_v, acc_sh.at[dst_v], add=True)
                if with_counts:
                    pltpu.sync_copy(ones_v, cnt_sh.at[dst_v], add=True)

        @pl.when(c == 0)
        def _():
            edge_loop(hlo_hbm, True)

        @pl.when(c == 1)
        def _():
            edge_loop(hhi_hbm, False)

        plsc.subcore_barrier()

        r0 = s * rows_per_tile
        pltpu.sync_copy(acc_sh.at[pl.ds(r0, rows_per_tile)],
                        acc_hbm.at[c, pl.ds(r0, rows_per_tile)])

        @pl.when(c == 0)
        def _():
            pltpu.sync_copy(cnt_sh.at[pl.ds(r0, rows_per_tile)],
                            cnt_hbm.at[pl.ds(r0, rows_per_tile)])

    return agg(h_lo, h_hi, src, dst)


def _tc_combine(h, acc, cnt, w1t, w2t, b2):
    n = h.shape[0]
    br = 1000
    grid = (n // br,)

    def body(h_ref, acc_ref, cnt_ref, w1_ref, w2_ref, b_ref, o_ref):
        a = jnp.concatenate([acc_ref[0], acc_ref[1]], axis=1)
        inv = 1.0 / jnp.maximum(cnt_ref[:, 0:1], 1.0)
        hn = a * inv
        t1 = jnp.dot(h_ref[...], w1_ref[...], preferred_element_type=jnp.float32)
        t2 = jnp.dot(hn, w2_ref[...], preferred_element_type=jnp.float32)
        o_ref[...] = t1 + t2 + b_ref[...]

    return pl.pallas_call(
        body,
        grid=grid,
        in_specs=[
            pl.BlockSpec((br, F), lambda i: (i, 0)),
            pl.BlockSpec((N_CORES, br, FH), lambda i: (0, i, 0)),
            pl.BlockSpec((br, CNT_W), lambda i: (i, 0)),
            pl.BlockSpec((F, F), lambda i: (0, 0)),
            pl.BlockSpec((F, F), lambda i: (0, 0)),
            pl.BlockSpec((1, F), lambda i: (0, 0)),
        ],
        out_specs=pl.BlockSpec((br, F), lambda i: (i, 0)),
        out_shape=jax.ShapeDtypeStruct((n, F), jnp.float32),
    )(h, acc, cnt, w1t, w2t, b2)


def kernel(h, edge_index, W, b):
    src = edge_index[0]
    dst = edge_index[1]
    f_in = h.shape[1]
    h_lo = h[:, :FH]
    h_hi = h[:, FH:]
    w1t = W[:, :f_in].T
    w2t = W[:, f_in:].T
    b2 = b.reshape(1, -1)
    acc, cnt = _sc_aggregate(h_lo, h_hi, src, dst)
    return _tc_combine(h, acc, cnt, w1t, w2t, b2)


# speedup vs baseline: 8.1388x; 2.0627x over previous
import functools

import jax
import jax.numpy as jnp
from jax import lax
from jax.experimental import pallas as pl
from jax.experimental.pallas import tpu as pltpu
from jax.experimental.pallas import tpu_sc as plsc

N_CORES = 2
N_SUBCORES = 16
CHUNK = 128
F = 128
FH = F // 2
CNT_W = 16


def _sc_aggregate(h_lo, h_hi, src3, dst3, n_nodes, n_pad):
    n_chunks = src3.shape[1]
    rows_per_tile = n_pad // N_SUBCORES
    zrows = rows_per_tile // 5

    mesh = plsc.VectorSubcoreMesh(core_axis_name="c", subcore_axis_name="s")

    @functools.partial(
        pl.kernel,
        out_type=[
            jax.ShapeDtypeStruct((N_CORES, n_pad, FH), jnp.float32),
            jax.ShapeDtypeStruct((N_CORES, n_pad, CNT_W), jnp.float32),
        ],
        mesh=mesh,
        scratch_types=[
            pltpu.VMEM((n_chunks, CHUNK), jnp.int32),
            pltpu.VMEM((n_chunks, CHUNK), jnp.int32),
            pltpu.VMEM((CHUNK, FH), jnp.float32),
            pltpu.VMEM((CHUNK, FH), jnp.float32),
            pltpu.VMEM((CHUNK, CNT_W), jnp.float32),
            pltpu.VMEM((zrows, FH), jnp.float32),
            pltpu.VMEM((zrows, CNT_W), jnp.float32),
            pltpu.VMEM_SHARED((n_pad, FH), jnp.float32),
            pltpu.VMEM_SHARED((n_pad, CNT_W), jnp.float32),
            pltpu.SemaphoreType.DMA,
            pltpu.SemaphoreType.DMA,
        ],
        compiler_params=pltpu.CompilerParams(use_tc_tiling_on_sc=False),
    )
    def agg(hlo_hbm, hhi_hbm, src_hbm, dst_hbm, acc_hbm, cnt_hbm,
            srcv, dstv, rows0, rows1, ones_v, zrow_v, zcnt_v,
            acc_sh, cnt_sh, sem0, sem1):
        c = lax.axis_index("c")
        s = lax.axis_index("s")

        pltpu.sync_copy(src_hbm.at[s], srcv)
        pltpu.sync_copy(dst_hbm.at[s], dstv)

        @pl.loop(0, CHUNK)
        def _(i):
            ones_v[i, :] = jnp.full((CNT_W,), 1.0, jnp.float32)

        @pl.loop(0, zrows)
        def _(i):
            for j in range(FH // 16):
                zrow_v[i, pl.ds(j * 16, 16)] = jnp.zeros((16,), jnp.float32)
            zcnt_v[i, :] = jnp.zeros((CNT_W,), jnp.float32)

        for j in range(rows_per_tile // zrows):
            r0 = s * rows_per_tile + j * zrows
            pltpu.sync_copy(zrow_v, acc_sh.at[pl.ds(r0, zrows)])
            pltpu.sync_copy(zcnt_v, cnt_sh.at[pl.ds(r0, zrows)])
        plsc.subcore_barrier()

        def run(h_half_hbm, parity):
            def fire(i, buf, sem):
                pltpu.async_copy(h_half_hbm.at[srcv.at[i]], buf, sem)

            def drain(i, buf, sem):
                pltpu.make_async_copy(h_half_hbm.at[srcv.at[i]], buf, sem).wait()

            def scat(i, buf, count):
                pltpu.sync_copy(buf, acc_sh.at[dstv.at[i]], add=True)
                if count:
                    pltpu.sync_copy(ones_v, cnt_sh.at[dstv.at[i]], add=True)

            fire(0, rows0, sem0)
            fire(1, rows1, sem1)

            @pl.loop(0, n_chunks - 2, step=2)
            def _(i):
                drain(i, rows0, sem0)
                scat(i, rows0, parity == 0)
                fire(i + 2, rows0, sem0)
                drain(i + 1, rows1, sem1)
                scat(i + 1, rows1, parity == 1)
                fire(i + 3, rows1, sem1)

            drain(n_chunks - 2, rows0, sem0)
            scat(n_chunks - 2, rows0, parity == 0)
            drain(n_chunks - 1, rows1, sem1)
            scat(n_chunks - 1, rows1, parity == 1)

        @pl.when(c == 0)
        def _():
            run(hlo_hbm, 0)

        @pl.when(c == 1)
        def _():
            run(hhi_hbm, 1)

        plsc.subcore_barrier()

        r0 = s * rows_per_tile
        pltpu.sync_copy(acc_sh.at[pl.ds(r0, rows_per_tile)],
                        acc_hbm.at[c, pl.ds(r0, rows_per_tile)])
        pltpu.sync_copy(cnt_sh.at[pl.ds(r0, rows_per_tile)],
                        cnt_hbm.at[c, pl.ds(r0, rows_per_tile)])

    return agg(h_lo, h_hi, src3, dst3)


def _tc_combine(h, acc, cnt, w1t, w2t, b2):
    n = h.shape[0]
    br = 1000
    grid = (n // br,)

    def body(h_ref, acc_ref, cnt_ref, w1_ref, w2_ref, b_ref, o_ref):
        a = jnp.concatenate([acc_ref[0], acc_ref[1]], axis=1)
        cn = cnt_ref[0, :, 0:1] + cnt_ref[1, :, 0:1]
        inv = 1.0 / jnp.maximum(cn, 1.0)
        hn = a * inv
        t1 = jnp.dot(h_ref[...], w1_ref[...], preferred_element_type=jnp.float32)
        t2 = jnp.dot(hn, w2_ref[...], preferred_element_type=jnp.float32)
        o_ref[...] = t1 + t2 + b_ref[...]

    return pl.pallas_call(
        body,
        grid=grid,
        in_specs=[
            pl.BlockSpec((br, F), lambda i: (i, 0)),
            pl.BlockSpec((N_CORES, br, FH), lambda i: (0, i, 0)),
            pl.BlockSpec((N_CORES, br, CNT_W), lambda i: (0, i, 0)),
            pl.BlockSpec((F, F), lambda i: (0, 0)),
            pl.BlockSpec((F, F), lambda i: (0, 0)),
            pl.BlockSpec((1, F), lambda i: (0, 0)),
        ],
        out_specs=pl.BlockSpec((br, F), lambda i: (i, 0)),
        out_shape=jax.ShapeDtypeStruct((n, F), jnp.float32),
    )(h, acc, cnt, w1t, w2t, b2)


def kernel(h, edge_index, W, b):
    n_nodes, f_in = h.shape
    n_edges = edge_index.shape[1]
    n_pad = ((n_nodes + 40 * N_SUBCORES - 1) // (40 * N_SUBCORES)) * 40 * N_SUBCORES

    per_tile = n_edges // N_SUBCORES
    n_chunks = -(-per_tile // CHUNK)
    if n_chunks % 2:
        n_chunks += 1
    pad = n_chunks * CHUNK - per_tile

    src = edge_index[0].reshape(N_SUBCORES, per_tile)
    dst = edge_index[1].reshape(N_SUBCORES, per_tile)
    if pad:
        pad_src = jnp.zeros((N_SUBCORES, pad), jnp.int32)
        spread = n_pad - n_nodes
        lanes = (jnp.arange(N_SUBCORES, dtype=jnp.int32)[:, None] * 37
                 + jnp.arange(pad, dtype=jnp.int32)[None, :])
        pad_dst = n_nodes + lanes % spread
        src = jnp.concatenate([src, pad_src], axis=1)
        dst = jnp.concatenate([dst, pad_dst], axis=1)
    src3 = src.reshape(N_SUBCORES, n_chunks, CHUNK)
    dst3 = dst.reshape(N_SUBCORES, n_chunks, CHUNK)

    h_lo = h[:, :FH]
    h_hi = h[:, FH:]
    w1t = W[:, :f_in].T
    w2t = W[:, f_in:].T
    b2 = b.reshape(1, -1)
    acc, cnt = _sc_aggregate(h_lo, h_hi, src3, dst3, n_nodes, n_pad)
    return _tc_combine(h, acc, cnt, w1t, w2t, b2)
